# baseline (speedup 1.0000x reference)
import functools

import jax
import jax.numpy as jnp
from jax import lax
from jax.experimental import pallas as pl
from jax.experimental.pallas import tpu as pltpu
from jax.experimental.pallas import tpu_sc as plsc

D = 128
SEQ = 200
B = 4096
HID = 300
NL = 2

NC, NS = 2, 16
NW = NC * NS
BPW = B // NW
CHUNKS = (104, 96)


def _sc_avg(indices, emb_table):
    mesh = plsc.VectorSubcoreMesh(core_axis_name="c", subcore_axis_name="s")

    @functools.partial(
        pl.kernel,
        mesh=mesh,
        compiler_params=pltpu.CompilerParams(
            use_tc_tiling_on_sc=False,
            disable_bounds_checks=True,
            disable_semaphore_checks=True,
        ),
        out_type=jax.ShapeDtypeStruct((B, D), jnp.float32),
        scratch_types=[
            pltpu.VMEM((2, 8, SEQ), jnp.int32),
            pltpu.VMEM((4, SEQ, D), jnp.float32),
            pltpu.VMEM((BPW, D), jnp.float32),
            [pltpu.SemaphoreType.DMA] * 4,
            [pltpu.SemaphoreType.DMA] * 2,
        ],
    )
    def k(idx_hbm, tbl_hbm, out_hbm, idx_v, rows_v, out_v, sem_rows, sem_idx):
        wid = lax.axis_index("s") * NC + lax.axis_index("c")
        base = wid * BPW

        def fetch_idx_block(g, jb):
            pltpu.async_copy(
                idx_hbm.at[pl.ds(base + g * 8, 8)], idx_v.at[jb], sem_idx[jb]
            )

        def wait_idx_block(jb):
            pltpu.make_async_copy(
                idx_hbm.at[pl.ds(0, 8)], idx_v.at[jb], sem_idx[jb]
            ).wait()

        def gather_row(b, j, jb, r):
            off = 0
            for ch in CHUNKS:
                pltpu.async_copy(
                    tbl_hbm.at[idx_v.at[jb, r, pl.ds(off, ch)]],
                    rows_v.at[j, pl.ds(off, ch)],
                    sem_rows[j],
                )
                off += ch

        def wait_row(j):
            pltpu.make_async_copy(
                tbl_hbm.at[pl.ds(0, SEQ)], rows_v.at[j], sem_rows[j]
            ).wait()

        fetch_idx_block(0, 0)
        fetch_idx_block(1, 1)
        wait_idx_block(0)
        for j in range(4):
            gather_row(j, j, 0, j)

        def body_oct(i, carry):
            for jb in range(2):
                g = 2 * i + jb
                for j in range(8):
                    b = 8 * g + j
                    wait_row(j % 4)

                    def body_s(si, acc):
                        accs = list(acc)
                        for u in range(8):
                            s = si * 8 + u
                            for d in range(8):
                                accs[d] = accs[d] + rows_v[j % 4, s, pl.ds(d * 16, 16)]
                        return tuple(accs)

                    acc = lax.fori_loop(
                        0, SEQ // 8, body_s,
                        tuple(jnp.zeros((16,), jnp.float32) for _ in range(8)),
                    )
                    scale = jnp.float32(1.0 / SEQ)
                    for d in range(8):
                        out_v[b, pl.ds(d * 16, 16)] = acc[d] * scale

                    if j == 4:
                        @pl.when(g + 2 < BPW // 8)
                        def _():
                            fetch_idx_block(g + 2, jb)

                        @pl.when(g + 1 < BPW // 8)
                        def _():
                            wait_idx_block(1 - jb)

                    @pl.when(b + 4 < BPW)
                    def _():
                        r_next = (j + 4) % 8
                        jb_next = jb if j < 4 else 1 - jb
                        gather_row(b + 4, j % 4, jb_next, r_next)
            return carry

        lax.fori_loop(0, BPW // 16, body_oct, 0)
        pltpu.sync_copy(out_v, out_hbm.at[pl.ds(base, BPW)])

    return k(indices, emb_table)


def _tc_mlp(avg, W1, b1, W2, b2):
    BT = 1024

    def body(avg_ref, w1_ref, b1_ref, w2_ref, b2_ref, out_ref):
        h = jnp.dot(avg_ref[...], w1_ref[...],
                    preferred_element_type=jnp.float32) + b1_ref[...]
        h = 0.5 * h * (1.0 + lax.erf(h * jnp.float32(0.7071067811865476)))
        out_ref[...] = jnp.dot(h, w2_ref[...],
                               preferred_element_type=jnp.float32) + b2_ref[...]

    return pl.pallas_call(
        body,
        grid=(B // BT,),
        in_specs=[
            pl.BlockSpec((BT, D), lambda i: (i, 0)),
            pl.BlockSpec((D, HID), lambda i: (0, 0)),
            pl.BlockSpec((1, HID), lambda i: (0, 0)),
            pl.BlockSpec((HID, NL), lambda i: (0, 0)),
            pl.BlockSpec((1, NL), lambda i: (0, 0)),
        ],
        out_specs=pl.BlockSpec((BT, NL), lambda i: (i, 0)),
        out_shape=jax.ShapeDtypeStruct((B, NL), jnp.float32),
    )(avg, W1, b1, W2, b2)


def kernel(indices, emb_table, W1, b1, W2, b2):
    avg = _sc_avg(indices.astype(jnp.int32), emb_table)
    return _tc_mlp(avg, W1, b1.reshape(1, HID), W2, b2.reshape(1, NL))
